# baseline (speedup 1.0000x reference)
import functools

import jax
import jax.numpy as jnp
from jax import lax
from jax.experimental import pallas as pl
from jax.experimental.pallas import tpu as pltpu
from jax.experimental.pallas import tpu_sc as plsc

HIDDEN = 64
IDX_MINOR = 128
GATHERS_PER_CHUNK = 8
CHUNK = IDX_MINOR * GATHERS_PER_CHUNK


@functools.partial(jax.jit, static_argnums=(2, 3))
def _sc_gather(ids2d, table, n_total, n_workers):
    b_per_w = n_total // n_workers
    n_chunks = b_per_w // CHUNK
    idx_rows_per_w = b_per_w // IDX_MINOR

    mesh = plsc.VectorSubcoreMesh(core_axis_name="c", subcore_axis_name="s")

    @functools.partial(
        pl.kernel,
        mesh=mesh,
        out_type=jax.ShapeDtypeStruct((n_total, HIDDEN), jnp.float32),
        scratch_types=[
            pltpu.VMEM((GATHERS_PER_CHUNK, IDX_MINOR), jnp.int32),
            pltpu.VMEM((CHUNK, HIDDEN), jnp.float32),
            pltpu.SemaphoreType.DMA,
        ],
        compiler_params=pltpu.CompilerParams(use_tc_tiling_on_sc=False),
    )
    def k(ids_hbm, table_hbm, out_hbm, idx_v, rows_v, sem):
        wid = lax.axis_index("s") * 2 + lax.axis_index("c")

        def chunk_body(c, carry):
            idx_row0 = wid * idx_rows_per_w + c * GATHERS_PER_CHUNK
            pltpu.sync_copy(ids_hbm.at[pl.ds(idx_row0, GATHERS_PER_CHUNK)],
                            idx_v)
            copies = []
            for j in range(GATHERS_PER_CHUNK):
                copies.append(pltpu.async_copy(
                    table_hbm.at[idx_v.at[j]],
                    rows_v.at[pl.ds(j * IDX_MINOR, IDX_MINOR)],
                    sem,
                ))
            for cp in copies:
                cp.wait()
            out0 = wid * b_per_w + c * CHUNK
            pltpu.sync_copy(rows_v, out_hbm.at[pl.ds(out0, CHUNK)])
            return carry

        lax.fori_loop(0, n_chunks, chunk_body, 0)

    return k(ids2d, table)


def kernel(input_ids, token_embedding):
    batch, seq = input_ids.shape
    n_total = batch * seq
    ids2d = input_ids.reshape(n_total // IDX_MINOR, IDX_MINOR).astype(jnp.int32)
    rows = _sc_gather(ids2d, token_embedding, n_total, 32)
    return rows.reshape(batch, seq, HIDDEN)


# speedup vs baseline: 1.0087x; 1.0087x over previous
import functools

import jax
import jax.numpy as jnp
from jax import lax
from jax.experimental import pallas as pl
from jax.experimental.pallas import tpu as pltpu
from jax.experimental.pallas import tpu_sc as plsc

HIDDEN = 64
IDX_MINOR = 128
G = 5
CHUNK = IDX_MINOR * G
NWORKERS = 32


@functools.partial(jax.jit, static_argnums=(2,))
def _sc_gather(ids3d, table, n_total):
    b_per_w = n_total // NWORKERS
    n_chunks = b_per_w // CHUNK

    mesh = plsc.VectorSubcoreMesh(core_axis_name="c", subcore_axis_name="s")

    @functools.partial(
        pl.kernel,
        mesh=mesh,
        out_type=jax.ShapeDtypeStruct((n_total, HIDDEN), jnp.float32),
        scratch_types=[
            pltpu.VMEM((G, IDX_MINOR), jnp.int32),
            pltpu.VMEM((G, IDX_MINOR), jnp.int32),
            pltpu.VMEM((CHUNK, HIDDEN), jnp.float32),
            pltpu.VMEM((CHUNK, HIDDEN), jnp.float32),
            pltpu.SemaphoreType.DMA,
            pltpu.SemaphoreType.DMA,
            pltpu.SemaphoreType.DMA,
            pltpu.SemaphoreType.DMA,
        ],
        compiler_params=pltpu.CompilerParams(use_tc_tiling_on_sc=False),
    )
    def k(ids_hbm, table_hbm, out_hbm, idx0, idx1, rows0, rows1,
          sg0, sg1, sw0, sw1):
        wid = lax.axis_index("s") * 2 + lax.axis_index("c")
        idx_v = (idx0, idx1)
        rows_v = (rows0, rows1)
        sem_g = (sg0, sg1)
        sem_w = (sw0, sw1)
        chunk0 = wid * n_chunks
        out0 = wid * b_per_w

        def load_and_gather(c, b):
            pltpu.sync_copy(ids_hbm.at[chunk0 + c], idx_v[b])
            for j in range(G):
                pltpu.async_copy(
                    table_hbm.at[idx_v[b].at[j]],
                    rows_v[b].at[pl.ds(j * IDX_MINOR, IDX_MINOR)],
                    sem_g[b],
                )

        def gather_drain(b):
            pltpu.make_async_copy(
                out_hbm.at[pl.ds(0, CHUNK)], rows_v[b], sem_g[b]).wait()

        def write_start(c, b):
            pltpu.async_copy(
                rows_v[b], out_hbm.at[pl.ds(out0 + c * CHUNK, CHUNK)],
                sem_w[b])

        def write_drain(b):
            pltpu.make_async_copy(
                out_hbm.at[pl.ds(0, CHUNK)], rows_v[b], sem_w[b]).wait()

        load_and_gather(0, 0)
        load_and_gather(1, 1)
        gather_drain(0)
        write_start(0, 0)

        def body(i, carry):
            for b in range(2):
                c = 2 * i + 2 + b
                write_drain(b)
                load_and_gather(c, b)
                gather_drain(1 - b)
                write_start(c - 1, 1 - b)
            return carry

        lax.fori_loop(0, (n_chunks - 2) // 2, body, 0)

        last_b = (n_chunks - 1) % 2
        gather_drain(last_b)
        write_start(n_chunks - 1, last_b)
        write_drain(1 - last_b)
        write_drain(last_b)

    return k(ids3d, table)


def kernel(input_ids, token_embedding):
    batch, seq = input_ids.shape
    n_total = batch * seq
    ids3d = input_ids.reshape(n_total // CHUNK, G, IDX_MINOR).astype(jnp.int32)
    rows = _sc_gather(ids3d, token_embedding, n_total)
    return rows.reshape(batch, seq, HIDDEN)


# speedup vs baseline: 1.0326x; 1.0237x over previous
import functools

import jax
import jax.numpy as jnp
from jax import lax
from jax.experimental import pallas as pl
from jax.experimental.pallas import tpu as pltpu
from jax.experimental.pallas import tpu_sc as plsc

HIDDEN = 64
IDX_MINOR = 128
G = 5
NWORKERS = 32


@jax.jit
def _sc_gather(ids_t, table):
    seq, batch = ids_t.shape
    bblk = batch // NWORKERS
    n_chunks = seq // G

    mesh = plsc.VectorSubcoreMesh(core_axis_name="c", subcore_axis_name="s")

    @functools.partial(
        pl.kernel,
        mesh=mesh,
        out_type=jax.ShapeDtypeStruct((seq, batch, HIDDEN), jnp.float32),
        scratch_types=[
            pltpu.VMEM((G, IDX_MINOR), jnp.int32),
            pltpu.VMEM((G, IDX_MINOR), jnp.int32),
            pltpu.VMEM((G, IDX_MINOR, HIDDEN), jnp.float32),
            pltpu.VMEM((G, IDX_MINOR, HIDDEN), jnp.float32),
            pltpu.SemaphoreType.DMA,
            pltpu.SemaphoreType.DMA,
            pltpu.SemaphoreType.DMA,
            pltpu.SemaphoreType.DMA,
        ],
        compiler_params=pltpu.CompilerParams(use_tc_tiling_on_sc=False),
    )
    def k(ids_hbm, table_hbm, out_hbm, idx0, idx1, rows0, rows1,
          sg0, sg1, sw0, sw1):
        wid = lax.axis_index("s") * 2 + lax.axis_index("c")
        idx_v = (idx0, idx1)
        rows_v = (rows0, rows1)
        sem_g = (sg0, sg1)
        sem_w = (sw0, sw1)
        col0 = wid * bblk

        def load_and_gather(c, b):
            pltpu.sync_copy(
                ids_hbm.at[pl.ds(c * G, G), pl.ds(col0, IDX_MINOR)],
                idx_v[b])
            for j in range(G):
                pltpu.async_copy(
                    table_hbm.at[idx_v[b].at[j]],
                    rows_v[b].at[j],
                    sem_g[b],
                )

        def gather_drain(b):
            pltpu.make_async_copy(
                out_hbm.at[pl.ds(0, G), pl.ds(0, IDX_MINOR)],
                rows_v[b], sem_g[b]).wait()

        def write_start(c, b):
            pltpu.async_copy(
                rows_v[b],
                out_hbm.at[pl.ds(c * G, G), pl.ds(col0, IDX_MINOR)],
                sem_w[b])

        def write_drain(b):
            pltpu.make_async_copy(
                out_hbm.at[pl.ds(0, G), pl.ds(0, IDX_MINOR)],
                rows_v[b], sem_w[b]).wait()

        load_and_gather(0, 0)
        load_and_gather(1, 1)
        gather_drain(0)
        write_start(0, 0)

        def body(i, carry):
            for b in range(2):
                c = 2 * i + 2 + b
                write_drain(b)
                load_and_gather(c, b)
                gather_drain(1 - b)
                write_start(c - 1, 1 - b)
            return carry

        lax.fori_loop(0, (n_chunks - 2) // 2, body, 0)

        last_b = (n_chunks - 1) % 2
        gather_drain(last_b)
        write_start(n_chunks - 1, last_b)
        write_drain(1 - last_b)
        write_drain(last_b)

    return k(ids_t, table)


def kernel(input_ids, token_embedding):
    ids_t = input_ids.T.astype(jnp.int32)
    rows = _sc_gather(ids_t, token_embedding)
    return rows.transpose(1, 0, 2)
